# Initial kernel scaffold; baseline (speedup 1.0000x reference)
import jax
import jax.numpy as jnp
from jax.experimental import pallas as pl


def kernel(x, edge_index, W, b):
    raise NotImplementedError("write your pallas kernel here")


# speedup vs baseline: 8.2645x; 8.2645x over previous
import jax
import jax.numpy as jnp
from jax import lax
from jax.experimental import pallas as pl
from jax.experimental.pallas import tpu as pltpu
from jax.experimental.pallas import tpu_sc as plsc

N = 10000
NPAD = 10240
E = 160000
CIN = 256
NC = 2
NQ = 2
CQ = CIN // (NC * NQ)
NS = 16
L = 16
EPW = E // NS
G = 80
NJ = EPW // G
BINS = NPAD // NS
BM = 1000

_GDN = lax.GatherDimensionNumbers(
    offset_dims=(), collapsed_slice_dims=(0,), start_index_map=(0,))


def _splat(v, r):
    idx = jnp.full((L, 1), r, dtype=jnp.int32)
    return lax.gather(v, idx, _GDN, (1,),
                      mode=lax.GatherScatterMode.PROMISE_IN_BOUNDS)


def _rsqrt(d):
    i = plsc.bitcast(d, jnp.int32)
    i = jnp.int32(0x5F3759DF) - (i >> 1)
    y = plsc.bitcast(i, jnp.float32)
    for _ in range(3):
        y = y * (1.5 - 0.5 * d * y * y)
    return y


def _sc_body(x_hbm, src_hbm, dst_hbm, u_hbm, w_hbm, y_hbm,
             src_my, dst_my, hist, hblk, dloc, dinv_v, rows, buf16,
             acc, hstage, dstage):
    cc = lax.axis_index("c")
    ss = lax.axis_index("s")
    zero16f = jnp.zeros((L,), jnp.float32)
    one16f = jnp.ones((L,), jnp.float32)

    pltpu.sync_copy(src_hbm.at[ss], src_my)
    pltpu.sync_copy(dst_hbm.at[ss], dst_my)

    def _adjust_src(off):
        off16 = jnp.full((L,), off, dtype=jnp.int32)

        def _adj(j, carry):
            for k in range(G // L):
                sl = (j, pl.ds(k * L, L))
                src_my[sl] = src_my[sl] + off16
            return carry
        lax.fori_loop(0, NJ, _adj, 0)

    _adjust_src(cc * (NQ * N))

    def _zero(i, carry):
        hist[pl.ds(i * L, L)] = zero16f
        return carry
    lax.fori_loop(0, NPAD // L, _zero, 0)

    def _hist(j, carry):
        for k in range(G // L):
            d = dst_my[j, pl.ds(k * L, L)]
            plsc.addupdate_scatter(hist, [d], one16f)
        return carry
    lax.fori_loop(0, NJ, _hist, 0)

    pltpu.sync_copy(hist, hstage.at[ss])
    plsc.subcore_barrier()
    for t in range(NS):
        pltpu.sync_copy(hstage.at[t, pl.ds(ss * BINS, BINS)], hblk.at[t])

    def _dinv(g, carry):
        acc16 = one16f
        for t in range(NS):
            acc16 = acc16 + hblk[t, pl.ds(g * L, L)]
        dloc[pl.ds(g * L, L)] = _rsqrt(acc16)
        return carry
    lax.fori_loop(0, BINS // L, _dinv, 0)
    pltpu.sync_copy(dloc, dstage.at[pl.ds(ss * BINS, BINS)])
    plsc.subcore_barrier()
    pltpu.sync_copy(dstage, dinv_v)

    row0 = ss * BINS
    ng = jnp.minimum(N - row0, BINS) // L

    def _scale_buf(f16):
        for r in range(L):
            f = _splat(f16, r)
            for k in range(CQ // L):
                sl = (r, pl.ds(k * L, L))
                buf16[sl] = buf16[sl] * f

    def _hop(table):
        def _j(j, carry):
            pltpu.sync_copy(table.at[src_my.at[j]], rows)
            pltpu.sync_copy(rows, acc.at[dst_my.at[j]], add=True)
            return carry
        lax.fori_loop(0, NJ, _j, 0)

    for q in range(NQ):
        qoff = (cc * NQ + q) * N
        col0 = cc * (NQ * CQ) + q * CQ

        def _p3(g, carry):
            r0 = row0 + g * L
            pltpu.sync_copy(x_hbm.at[pl.ds(r0, L), pl.ds(col0, CQ)], buf16)
            _scale_buf(dinv_v[pl.ds(r0, L)])
            pltpu.sync_copy(buf16, u_hbm.at[pl.ds(qoff + r0, L)])
            pltpu.sync_copy(buf16, acc.at[pl.ds(r0, L)])
            return carry
        lax.fori_loop(0, ng, _p3, 0)
        plsc.subcore_barrier()

        _hop(u_hbm)
        plsc.subcore_barrier()

        def _p6(g, carry):
            r0 = row0 + g * L
            pltpu.sync_copy(acc.at[pl.ds(r0, L)], buf16)
            dv = dinv_v[pl.ds(r0, L)]
            _scale_buf(dv * dv)
            pltpu.sync_copy(buf16, w_hbm.at[pl.ds(qoff + r0, L)])
            pltpu.sync_copy(buf16, acc.at[pl.ds(r0, L)])
            return carry
        lax.fori_loop(0, ng, _p6, 0)
        plsc.subcore_barrier()

        _hop(w_hbm)
        plsc.subcore_barrier()

        def _p8(g, carry):
            r0 = row0 + g * L
            pltpu.sync_copy(acc.at[pl.ds(r0, L)], buf16)
            _scale_buf(dinv_v[pl.ds(r0, L)])
            pltpu.sync_copy(buf16, y_hbm.at[pl.ds(qoff + r0, L)])
            return carry
        lax.fori_loop(0, ng, _p8, 0)

        if q + 1 < NQ:
            _adjust_src(N)


def _mm_body(y0_ref, y1_ref, y2_ref, y3_ref,
             w0_ref, w1_ref, w2_ref, w3_ref, b_ref, o_ref):
    o_ref[...] = (
        jnp.dot(y0_ref[...], w0_ref[...], preferred_element_type=jnp.float32)
        + jnp.dot(y1_ref[...], w1_ref[...], preferred_element_type=jnp.float32)
        + jnp.dot(y2_ref[...], w2_ref[...], preferred_element_type=jnp.float32)
        + jnp.dot(y3_ref[...], w3_ref[...], preferred_element_type=jnp.float32)
        + b_ref[...])


def kernel(x, edge_index, W, b):
    ei = edge_index.astype(jnp.int32)
    srcr = ei[0].reshape(NS, NJ, G)
    dstr = ei[1].reshape(NS, NJ, G)

    mesh = plsc.VectorSubcoreMesh(core_axis_name="c", subcore_axis_name="s")
    out_t = (jax.ShapeDtypeStruct((NC * NQ * N, CQ), jnp.float32),) * 3
    scratch = [
        pltpu.VMEM((NJ, G), jnp.int32),
        pltpu.VMEM((NJ, G), jnp.int32),
        pltpu.VMEM((NPAD,), jnp.float32),
        pltpu.VMEM((NS, BINS), jnp.float32),
        pltpu.VMEM((BINS,), jnp.float32),
        pltpu.VMEM((NPAD,), jnp.float32),
        pltpu.VMEM((G, CQ), jnp.float32),
        pltpu.VMEM((L, CQ), jnp.float32),
        pltpu.VMEM_SHARED((NPAD, CQ), jnp.float32),
        pltpu.VMEM_SHARED((NS, NPAD), jnp.float32),
        pltpu.VMEM_SHARED((NPAD,), jnp.float32),
    ]
    sc = pl.kernel(_sc_body, out_type=out_t, mesh=mesh, scratch_types=scratch,
                   compiler_params=pltpu.CompilerParams(needs_layout_passes=False, use_tc_tiling_on_sc=False))
    _, _, y = sc(x, srcr, dstr)

    nb = N // BM
    out = pl.pallas_call(
        _mm_body,
        grid=(nb,),
        in_specs=[
            pl.BlockSpec((BM, CQ), lambda i: (i, 0)),
            pl.BlockSpec((BM, CQ), lambda i: (i + nb, 0)),
            pl.BlockSpec((BM, CQ), lambda i: (i + 2 * nb, 0)),
            pl.BlockSpec((BM, CQ), lambda i: (i + 3 * nb, 0)),
            pl.BlockSpec((CQ, CIN), lambda i: (0, 0)),
            pl.BlockSpec((CQ, CIN), lambda i: (0, 0)),
            pl.BlockSpec((CQ, CIN), lambda i: (0, 0)),
            pl.BlockSpec((CQ, CIN), lambda i: (0, 0)),
            pl.BlockSpec((1, CIN), lambda i: (0, 0)),
        ],
        out_specs=pl.BlockSpec((BM, CIN), lambda i: (i, 0)),
        out_shape=jax.ShapeDtypeStruct((N, CIN), jnp.float32),
    )(y, y, y, y, W[:CQ], W[CQ:2 * CQ], W[2 * CQ:3 * CQ], W[3 * CQ:],
      b.reshape(1, CIN))
    return out
